# Initial kernel scaffold; baseline (speedup 1.0000x reference)
import jax
import jax.numpy as jnp
from jax.experimental import pallas as pl


def kernel(atom_embed, atom_enc_w, atom_enc_b, bond_type_embed, valence_embed, valence_enc_w, valence_enc_b, final_enc_w, final_enc_b, lat_fc_w, lat_fc_scale, lat_fc_shift, gru0_wih, gru0_whh, gru0_bih, gru0_bhh, gru1_wih, gru1_whh, gru1_bih, gru1_bhh, attn0_wqkv, attn0_bqkv, attn0_wo, attn0_bo, bp_wl, bp_wr, bp_wo, val_out_w, val_out_b, atom_out_w, atom_out_b, z, atypes, bond_types, bonded_atoms, atom_valences, adj):
    raise NotImplementedError("write your pallas kernel here")


# speedup vs baseline: 32.4125x; 32.4125x over previous
import functools

import jax
import jax.numpy as jnp
from jax.experimental import pallas as pl
from jax.experimental.pallas import tpu as pltpu

T = 8
H = 32
NB = 5
NA = 16
NV = 5
F = 16
LAT = 32
FEN = 32
AE, VE = 16, 8
AEN, VEN = 32, 16
AV_IN = AE + VE
BEMB = 32
DH = 16
LEAK = 0.1

BR_AV, BR_F, BR_LAT, BR_IH0, BR_HH0, BR_IH1, BR_HH1, BR_QKV, BR_O, BR_HEAD = range(10)


def _leaky(x):
    return jnp.where(x > 0, x, LEAK * x)


def _dot(a, b):
    return jax.lax.dot_general(a, b, (((1,), (0,)), ((), ())),
                               preferred_element_type=jnp.float32)


def _body(avin_ref, bemb_ref, z_ref, adj_ref,
          wav_ref, wfav_ref, wfb_ref, latw_ref,
          wih0l_ref, wih0e_ref, whh0_ref, wih1_ref, whh1_ref,
          wqkv_ref, wo_ref, whead_ref, wbd_ref, b_ref,
          oav_ref, obond_ref, *, m):
    tm = T * m

    def bias(row, width):
        return b_ref[row:row + 1, 0:width]

    avin = avin_ref[...].reshape(tm, AV_IN)
    av_pre = _leaky(_dot(avin, wav_ref[...]) + bias(BR_AV, 96))
    av3 = av_pre.reshape(T, m, 96)
    adjc = adj_ref[...]

    def adj_apply(x3):
        out = []
        for t in range(T):
            acc = adjc[:, t * 8:t * 8 + 1] * x3[0]
            for j in range(1, T):
                acc = acc + adjc[:, t * 8 + j:t * 8 + j + 1] * x3[j]
            out.append(acc)
        return out

    avenc = jnp.concatenate(adj_apply(av3), axis=0)
    bembf = bemb_ref[...].reshape(tm, BEMB)
    enc_pre = _leaky(_dot(avenc, wfav_ref[...]) + _dot(bembf, wfb_ref[...])
                     + bias(BR_F, FEN))
    enc_list = adj_apply(enc_pre.reshape(T, m, FEN))
    enc_all = jnp.concatenate(enc_list, axis=0)

    lat = _leaky(_dot(z_ref[...], latw_ref[...]) + bias(BR_LAT, 32))
    gi0c = _dot(lat, wih0l_ref[...]) + bias(BR_IH0, 96)
    gi0 = _dot(enc_all, wih0e_ref[...]).reshape(T, m, 96)

    def gru(gis, whh_ref2, bhh_row):
        whh = whh_ref2[...]
        bhh = bias(bhh_row, 96)
        h = jnp.zeros((m, H), jnp.float32)
        outs = []
        for t in range(T):
            gh = _dot(h, whh) + bhh
            git = gis[t]
            rz = jax.nn.sigmoid(git[:, :2 * H] + gh[:, :2 * H])
            r = rz[:, :H]
            zg = rz[:, H:]
            n = jnp.tanh(git[:, 2 * H:] + r * gh[:, 2 * H:])
            h = (1.0 - zg) * n + zg * h
            outs.append(h)
        return outs

    x1 = gru([gi0[t] + gi0c for t in range(T)], whh0_ref, BR_HH0)
    gi1 = (_dot(jnp.concatenate(x1, axis=0), wih1_ref[...])
           + bias(BR_IH1, 96)).reshape(T, m, 96)
    dec = gru([gi1[t] for t in range(T)], whh1_ref, BR_HH1)

    qkv = (_dot(jnp.concatenate(dec, axis=0), wqkv_ref[...])
           + bias(BR_QKV, 96)).reshape(T, m, 96)
    scale = 1.0 / float(DH) ** 0.5
    q = [qkv[t][:, 0:H] * scale for t in range(T)]
    k = [qkv[t][:, H:2 * H] for t in range(T)]
    v = [qkv[t][:, 2 * H:3 * H] for t in range(T)]
    kcat = jnp.concatenate(k, axis=1)

    rr = jax.lax.broadcasted_iota(jnp.int32, (T * H, 128), 0)
    cc = jax.lax.broadcasted_iota(jnp.int32, (T * H, 128), 1)
    sel = jnp.where(cc == ((rr % H) // DH) * 8 + rr // H, 1.0, 0.0)

    ccol = jax.lax.broadcasted_iota(jnp.int32, (1, 128), 1)
    att = []
    for i in range(T):
        kw = (i + 1) * H
        prod = jnp.concatenate([q[i]] * (i + 1), axis=1) * kcat[:, :kw]
        s = _dot(prod, sel[:kw, :])
        neg = jnp.where((ccol < 16) & ((ccol % 8) <= i), 0.0, -1e30)
        e = jnp.exp(s + neg)
        s0 = jnp.sum(e[:, 0:8], axis=1, keepdims=True)
        s1 = jnp.sum(e[:, 8:16], axis=1, keepdims=True)
        acc = None
        for j in range(i + 1):
            wj = jnp.concatenate([jnp.broadcast_to(e[:, j:j + 1], (m, DH)),
                                  jnp.broadcast_to(e[:, 8 + j:9 + j], (m, DH))],
                                 axis=1)
            term = wj * v[j]
            acc = term if acc is None else acc + term
        rcat = jnp.concatenate([jnp.broadcast_to(1.0 / s0, (m, DH)),
                                jnp.broadcast_to(1.0 / s1, (m, DH))], axis=1)
        att.append(acc * rcat)

    o = _dot(jnp.concatenate(att, axis=0), wo_ref[...]) + bias(BR_O, H)
    head = _dot(o, whead_ref[...]) + bias(BR_HEAD, 128)
    h3 = head.reshape(T, m, 128)
    oav_ref[...] = h3[:, :, 0:24]

    wbd = wbd_ref[...]
    for i in range(T):
        li = h3[i][:, 32:48]
        pairs = jnp.concatenate([_leaky(li + h3[j][:, 48:64]) for j in range(T)],
                                axis=1)
        obond_ref[i] = _dot(pairs, wbd)


def kernel(atom_embed, atom_enc_w, atom_enc_b, bond_type_embed, valence_embed,
           valence_enc_w, valence_enc_b, final_enc_w, final_enc_b, lat_fc_w,
           lat_fc_scale, lat_fc_shift, gru0_wih, gru0_whh, gru0_bih, gru0_bhh,
           gru1_wih, gru1_whh, gru1_bih, gru1_bhh, attn0_wqkv, attn0_bqkv,
           attn0_wo, attn0_bo, bp_wl, bp_wr, bp_wo, val_out_w, val_out_b,
           atom_out_w, atom_out_b, z, atypes, bond_types, bonded_atoms,
           atom_valences, adj):
    f32 = jnp.float32
    B = z.shape[0]
    m = 256 if B % 256 == 0 else B

    aemb = atom_embed[atypes.T]
    vemb = valence_embed[atom_valences.T]
    avin = jnp.concatenate([aemb, vemb], axis=2)
    bemb = bond_type_embed[bond_types.transpose(1, 0, 2)].reshape(T, B, BEMB)
    adj2 = adj.reshape(B, T * T)

    wav = (jnp.zeros((AV_IN, 96), f32)
           .at[:AE, :AEN].set(atom_enc_w)
           .at[AE:, AEN:AEN + VEN].set(valence_enc_w))
    wfav = (jnp.zeros((96, FEN), f32)
            .at[:AEN].set(final_enc_w[:AEN])
            .at[AEN:AEN + VEN].set(final_enc_w[AEN + BEMB:]))
    wfb = final_enc_w[AEN:AEN + BEMB]
    latw = lat_fc_w * lat_fc_scale[None, :]
    whead = (jnp.zeros((H, 128), f32)
             .at[:, 0:NA].set(atom_out_w)
             .at[:, NA:NA + NV].set(val_out_w)
             .at[:, 32:48].set(bp_wl)
             .at[:, 48:64].set(bp_wr))
    wbd = jnp.zeros((T * F, 40), f32)
    for j in range(T):
        wbd = wbd.at[j * F:(j + 1) * F, j * NB:(j + 1) * NB].set(bp_wo)

    bias = jnp.zeros((16, 128), f32)
    bias = bias.at[BR_AV, 0:AEN].set(atom_enc_b)
    bias = bias.at[BR_AV, AEN:AEN + VEN].set(valence_enc_b)
    bias = bias.at[BR_F, 0:FEN].set(final_enc_b)
    bias = bias.at[BR_LAT, 0:32].set(lat_fc_shift)
    bias = bias.at[BR_IH0, 0:96].set(gru0_bih)
    bias = bias.at[BR_HH0, 0:96].set(gru0_bhh)
    bias = bias.at[BR_IH1, 0:96].set(gru1_bih)
    bias = bias.at[BR_HH1, 0:96].set(gru1_bhh)
    bias = bias.at[BR_QKV, 0:96].set(attn0_bqkv)
    bias = bias.at[BR_O, 0:H].set(attn0_bo)
    bias = bias.at[BR_HEAD, 0:NA].set(atom_out_b)
    bias = bias.at[BR_HEAD, NA:NA + NV].set(val_out_b)

    body = functools.partial(_body, m=m)

    def wfull(shape):
        return pl.BlockSpec(shape, lambda i: (0, 0))

    oav, obond = pl.pallas_call(
        body,
        grid=(B // m,),
        in_specs=[
            pl.BlockSpec((T, m, AV_IN), lambda i: (0, i, 0)),
            pl.BlockSpec((T, m, BEMB), lambda i: (0, i, 0)),
            pl.BlockSpec((m, LAT), lambda i: (i, 0)),
            pl.BlockSpec((m, T * T), lambda i: (i, 0)),
            wfull((AV_IN, 96)),
            wfull((96, FEN)),
            wfull((BEMB, FEN)),
            wfull((LAT, 32)),
            wfull((32, 96)),
            wfull((32, 96)),
            wfull((32, 96)),
            wfull((32, 96)),
            wfull((32, 96)),
            wfull((32, 96)),
            wfull((32, 32)),
            wfull((H, 128)),
            wfull((T * F, 40)),
            wfull((16, 128)),
        ],
        out_specs=[pl.BlockSpec((T, m, 24), lambda i: (0, i, 0)),
                   pl.BlockSpec((T, m, 40), lambda i: (0, i, 0))],
        out_shape=(jax.ShapeDtypeStruct((T, B, 24), f32),
                   jax.ShapeDtypeStruct((T, B, 40), f32)),
        compiler_params=pltpu.CompilerParams(dimension_semantics=("parallel",)),
    )(avin, bemb, z, adj2, wav, wfav, wfb, latw,
      gru0_wih[:32], gru0_wih[32:], gru0_whh, gru1_wih, gru1_whh,
      attn0_wqkv, attn0_wo, whead, wbd, bias)

    av = oav.transpose(1, 0, 2)
    bonds = obond.transpose(1, 0, 2).reshape(B, T, T, NB)
    return {"atom_types": av[:, :, 0:NA],
            "bonds": bonds,
            "atom_valences": av[:, :, NA:NA + NV],
            "kps_1h": None}


# speedup vs baseline: 123.2992x; 3.8041x over previous
import functools

import jax
import jax.numpy as jnp
from jax.experimental import pallas as pl
from jax.experimental.pallas import tpu as pltpu

T = 8
H = 32
NB = 5
NA = 16
NV = 5
F = 16
LAT = 32
FEN = 32
AEN, VEN = 32, 16
BEMB = 32
DH = 16
LEAK = 0.1

BR_AV, BR_F, BR_LAT, BR_IH0, BR_HH0, BR_IH1, BR_HH1, BR_QKV, BR_O, BR_HEAD = range(10)


def _leaky(x):
    return jnp.where(x > 0, x, LEAK * x)


def _dot(a, b):
    return jax.lax.dot_general(a, b, (((1,), (0,)), ((), ())),
                               preferred_element_type=jnp.float32)


def _body(at_ref, bt_ref, va_ref, z_ref, adj_ref,
          wav2_ref, wfav_ref, wfb2_ref, latw_ref,
          wih0l_ref, wih0e_ref, whh0_ref, wih1_ref, whh1_ref,
          wqkv_ref, wo_ref, whead_ref, wbd_ref, b_ref,
          oav_ref, obond_ref, *, m):
    tm = T * m

    def bias(row, width):
        return b_ref[row:row + 1, 0:width]

    cc16 = jax.lax.broadcasted_iota(jnp.int32, (m, NA), 1)
    cc5 = jax.lax.broadcasted_iota(jnp.int32, (m, NV), 1)
    oh_av, oh_bd = [], []
    for t in range(T):
        ia = at_ref[:, t:t + 1]
        iv = va_ref[:, t:t + 1]
        oh_a = jnp.where(cc16 == ia, 1.0, 0.0)
        oh_v = jnp.where(cc5 == iv, 1.0, 0.0)
        oh_av.append(jnp.concatenate([oh_a, oh_v], axis=1))
        oh_bd.append(jnp.concatenate(
            [jnp.where(cc5 == bt_ref[:, 4 * t + s:4 * t + s + 1], 1.0, 0.0)
             for s in range(4)], axis=1))
    ohav = jnp.concatenate(oh_av, axis=0)
    ohbd = jnp.concatenate(oh_bd, axis=0)

    av_pre = _leaky(_dot(ohav, wav2_ref[...]) + bias(BR_AV, 96))
    av3 = av_pre.reshape(T, m, 96)
    adjc = adj_ref[...]

    def adj_apply(x3):
        out = []
        for t in range(T):
            acc = adjc[:, t * 8:t * 8 + 1] * x3[0]
            for j in range(1, T):
                acc = acc + adjc[:, t * 8 + j:t * 8 + j + 1] * x3[j]
            out.append(acc)
        return out

    avenc = jnp.concatenate(adj_apply(av3), axis=0)
    enc_pre = _leaky(_dot(avenc, wfav_ref[...]) + _dot(ohbd, wfb2_ref[...])
                     + bias(BR_F, FEN))
    enc_all = jnp.concatenate(adj_apply(enc_pre.reshape(T, m, FEN)), axis=0)

    lat = _leaky(_dot(z_ref[...], latw_ref[...]) + bias(BR_LAT, 32))
    gi0c = _dot(lat, wih0l_ref[...]) + bias(BR_IH0, 96)
    gi0 = _dot(enc_all, wih0e_ref[...]).reshape(T, m, 96)

    whh0 = whh0_ref[...]
    whh1 = whh1_ref[...]
    wih1 = wih1_ref[...]
    bhh0 = bias(BR_HH0, 96)
    bhh1 = bias(BR_HH1, 96)
    bih1 = bias(BR_IH1, 96)

    def cell(h, git, whh, bhh):
        gh = _dot(h, whh) + bhh
        rz = jax.nn.sigmoid(git[:, :2 * H] + gh[:, :2 * H])
        r = rz[:, :H]
        zg = rz[:, H:]
        n = jnp.tanh(git[:, 2 * H:] + r * gh[:, 2 * H:])
        return (1.0 - zg) * n + zg * h

    h1 = jnp.zeros((m, H), jnp.float32)
    h2 = jnp.zeros((m, H), jnp.float32)
    dec = []
    for t in range(T):
        h1 = cell(h1, gi0[t] + gi0c, whh0, bhh0)
        h2 = cell(h2, _dot(h1, wih1) + bih1, whh1, bhh1)
        dec.append(h2)

    decall = jnp.concatenate(dec, axis=0)
    qkv = _dot(decall, wqkv_ref[...]) + bias(BR_QKV, 96)
    scale = 1.0 / float(DH) ** 0.5
    qall = qkv[:, 0:H] * scale
    qkv3 = qkv.reshape(T, m, 96)
    kcat = jnp.concatenate([qkv3[t][:, H:2 * H] for t in range(T)], axis=1)
    vcat = jnp.concatenate([qkv3[t][:, 2 * H:3 * H] for t in range(T)], axis=1)
    kcat_rep = jnp.concatenate([kcat] * T, axis=0)
    vcat_rep = jnp.concatenate([vcat] * T, axis=0)
    qrep = jnp.concatenate([qall] * T, axis=1)

    rr = jax.lax.broadcasted_iota(jnp.int32, (T * H, 128), 0)
    cc = jax.lax.broadcasted_iota(jnp.int32, (T * H, 128), 1)
    sel = jnp.where(cc == ((rr % H) // DH) * 8 + rr // H, 1.0, 0.0)
    s_all = _dot(qrep * kcat_rep, sel)

    ccol = jax.lax.broadcasted_iota(jnp.int32, (1, 128), 1)
    mask = jnp.concatenate(
        [jnp.broadcast_to(
            jnp.where((ccol < 16) & ((ccol % 8) <= i), 1.0, 0.0), (m, 128))
         for i in range(T)], axis=0)
    e_all = jnp.exp(s_all) * mask
    s0 = jnp.sum(e_all[:, 0:8], axis=1, keepdims=True)
    s1 = jnp.sum(e_all[:, 8:16], axis=1, keepdims=True)

    rr2 = jax.lax.broadcasted_iota(jnp.int32, (DH, 256), 0)
    cc2 = jax.lax.broadcasted_iota(jnp.int32, (DH, 256), 1)
    expand = jnp.where(rr2 == ((cc2 % H) // DH) * 8 + cc2 // H, 1.0, 0.0)
    pexp = _dot(e_all[:, 0:DH], expand)
    rr3 = jax.lax.broadcasted_iota(jnp.int32, (T * H, H), 0)
    cc3 = jax.lax.broadcasted_iota(jnp.int32, (T * H, H), 1)
    sumsel = jnp.where(rr3 % H == cc3, 1.0, 0.0)
    ctxw = _dot(pexp * vcat_rep, sumsel)
    rcat = jnp.concatenate([jnp.broadcast_to(1.0 / s0, (tm, DH)),
                            jnp.broadcast_to(1.0 / s1, (tm, DH))], axis=1)
    att = ctxw * rcat

    o = _dot(att, wo_ref[...]) + bias(BR_O, H)
    head = _dot(o, whead_ref[...]) + bias(BR_HEAD, 128)
    oav_ref[...] = head.reshape(T, m, 128)[:, :, 0:24]

    lrep = jnp.concatenate([head[:, 32:48]] * T, axis=1)
    h3 = head.reshape(T, m, 128)
    rcat2 = jnp.concatenate([h3[j][:, 48:64] for j in range(T)], axis=1)
    rrep = jnp.concatenate([rcat2] * T, axis=0)
    pairs = _leaky(lrep + rrep)
    obond_ref[...] = _dot(pairs, wbd_ref[...]).reshape(T, m, 40)


def kernel(atom_embed, atom_enc_w, atom_enc_b, bond_type_embed, valence_embed,
           valence_enc_w, valence_enc_b, final_enc_w, final_enc_b, lat_fc_w,
           lat_fc_scale, lat_fc_shift, gru0_wih, gru0_whh, gru0_bih, gru0_bhh,
           gru1_wih, gru1_whh, gru1_bih, gru1_bhh, attn0_wqkv, attn0_bqkv,
           attn0_wo, attn0_bo, bp_wl, bp_wr, bp_wo, val_out_w, val_out_b,
           atom_out_w, atom_out_b, z, atypes, bond_types, bonded_atoms,
           atom_valences, adj):
    f32 = jnp.float32
    B = z.shape[0]
    m = 256 if B % 256 == 0 else B

    at_i = atypes.astype(jnp.int32)
    bt_i = bond_types.reshape(B, 32).astype(jnp.int32)
    va_i = atom_valences.astype(jnp.int32)
    adj2 = adj.reshape(B, T * T)

    wav = (jnp.zeros((24, 96), f32)
           .at[:16, :AEN].set(atom_enc_w)
           .at[16:, AEN:AEN + VEN].set(valence_enc_w))
    wemb_av = (jnp.zeros((21, 24), f32)
               .at[0:16, 0:16].set(atom_embed)
               .at[16:21, 16:24].set(valence_embed))
    wav2 = wemb_av @ wav
    wfav = (jnp.zeros((96, FEN), f32)
            .at[:AEN].set(final_enc_w[:AEN])
            .at[AEN:AEN + VEN].set(final_enc_w[AEN + BEMB:]))
    wfb = final_enc_w[AEN:AEN + BEMB]
    wemb_b = jnp.zeros((20, BEMB), f32)
    for s in range(4):
        wemb_b = wemb_b.at[5 * s:5 * (s + 1), 8 * s:8 * (s + 1)].set(bond_type_embed)
    wfb2 = wemb_b @ wfb
    latw = lat_fc_w * lat_fc_scale[None, :]
    whead = (jnp.zeros((H, 128), f32)
             .at[:, 0:NA].set(atom_out_w)
             .at[:, NA:NA + NV].set(val_out_w)
             .at[:, 32:48].set(bp_wl)
             .at[:, 48:64].set(bp_wr))
    wbd = jnp.zeros((T * F, 40), f32)
    for j in range(T):
        wbd = wbd.at[j * F:(j + 1) * F, j * NB:(j + 1) * NB].set(bp_wo)

    bias = jnp.zeros((16, 128), f32)
    bias = bias.at[BR_AV, 0:AEN].set(atom_enc_b)
    bias = bias.at[BR_AV, AEN:AEN + VEN].set(valence_enc_b)
    bias = bias.at[BR_F, 0:FEN].set(final_enc_b)
    bias = bias.at[BR_LAT, 0:32].set(lat_fc_shift)
    bias = bias.at[BR_IH0, 0:96].set(gru0_bih)
    bias = bias.at[BR_HH0, 0:96].set(gru0_bhh)
    bias = bias.at[BR_IH1, 0:96].set(gru1_bih)
    bias = bias.at[BR_HH1, 0:96].set(gru1_bhh)
    bias = bias.at[BR_QKV, 0:96].set(attn0_bqkv)
    bias = bias.at[BR_O, 0:H].set(attn0_bo)
    bias = bias.at[BR_HEAD, 0:NA].set(atom_out_b)
    bias = bias.at[BR_HEAD, NA:NA + NV].set(val_out_b)

    body = functools.partial(_body, m=m)

    def wfull(shape):
        return pl.BlockSpec(shape, lambda i: (0, 0))

    oav, obond = pl.pallas_call(
        body,
        grid=(B // m,),
        in_specs=[
            pl.BlockSpec((m, T), lambda i: (i, 0)),
            pl.BlockSpec((m, 4 * T), lambda i: (i, 0)),
            pl.BlockSpec((m, T), lambda i: (i, 0)),
            pl.BlockSpec((m, LAT), lambda i: (i, 0)),
            pl.BlockSpec((m, T * T), lambda i: (i, 0)),
            wfull((21, 96)),
            wfull((96, FEN)),
            wfull((20, FEN)),
            wfull((LAT, 32)),
            wfull((32, 96)),
            wfull((32, 96)),
            wfull((32, 96)),
            wfull((32, 96)),
            wfull((32, 96)),
            wfull((32, 96)),
            wfull((32, 32)),
            wfull((H, 128)),
            wfull((T * F, 40)),
            wfull((16, 128)),
        ],
        out_specs=[pl.BlockSpec((T, m, 24), lambda i: (0, i, 0)),
                   pl.BlockSpec((T, m, 40), lambda i: (0, i, 0))],
        out_shape=(jax.ShapeDtypeStruct((T, B, 24), f32),
                   jax.ShapeDtypeStruct((T, B, 40), f32)),
        compiler_params=pltpu.CompilerParams(dimension_semantics=("parallel",)),
    )(at_i, bt_i, va_i, z, adj2, wav2, wfav, wfb2, latw,
      gru0_wih[:32], gru0_wih[32:], gru0_whh, gru1_wih, gru1_whh,
      attn0_wqkv, attn0_wo, whead, wbd, bias)

    av = oav.transpose(1, 0, 2)
    bonds = obond.transpose(1, 0, 2).reshape(B, T, T, NB)
    return {"atom_types": av[:, :, 0:NA],
            "bonds": bonds,
            "atom_valences": av[:, :, NA:NA + NV],
            "kps_1h": None}


# speedup vs baseline: 141.6269x; 1.1486x over previous
import functools

import jax
import jax.numpy as jnp
from jax.experimental import pallas as pl
from jax.experimental.pallas import tpu as pltpu

T = 8
H = 32
NB = 5
NA = 16
NV = 5
F = 16
LAT = 32
FEN = 32
AEN, VEN = 32, 16
BEMB = 32
DH = 16
LEAK = 0.1

BR_AV, BR_F, BR_LAT, BR_IH0, BR_HH0, BR_IH1, BR_HH1, BR_QKV, BR_O, BR_HEAD = range(10)


def _leaky(x):
    return jnp.where(x > 0, x, LEAK * x)


def _dot(a, b):
    return jax.lax.dot_general(a, b, (((1,), (0,)), ((), ())),
                               preferred_element_type=jnp.float32)


def _body(at_ref, bt_ref, va_ref, z_ref, adj_ref,
          wav2_ref, wfav_ref, wfb2_ref, latw_ref,
          wih0l_ref, wih0e_ref, whh0_ref, wih1_ref, whh1_ref,
          wqkv_ref, wo_ref, whead_ref, wbd_ref, b_ref,
          oav_ref, obond_ref, *, m):
    tm = T * m

    def bias(row, width):
        return b_ref[row:row + 1, 0:width]

    cc16 = jax.lax.broadcasted_iota(jnp.int32, (m, NA), 1)
    cc5 = jax.lax.broadcasted_iota(jnp.int32, (m, NV), 1)
    oh_av, oh_bd = [], []
    for t in range(T):
        ia = at_ref[:, t:t + 1]
        iv = va_ref[:, t:t + 1]
        oh_a = jnp.where(cc16 == ia, 1.0, 0.0)
        oh_v = jnp.where(cc5 == iv, 1.0, 0.0)
        oh_av.append(jnp.concatenate([oh_a, oh_v], axis=1))
        oh_bd.append(jnp.concatenate(
            [jnp.where(cc5 == bt_ref[:, 4 * t + s:4 * t + s + 1], 1.0, 0.0)
             for s in range(4)], axis=1))
    ohav = jnp.concatenate(oh_av, axis=0)
    ohbd = jnp.concatenate(oh_bd, axis=0)

    av_pre = _leaky(_dot(ohav, wav2_ref[...]) + bias(BR_AV, 96))
    av3 = av_pre.reshape(T, m, 96)
    adjc = adj_ref[...]

    def adj_apply(x3):
        out = []
        for t in range(T):
            acc = adjc[:, t * 8:t * 8 + 1] * x3[0]
            for j in range(1, T):
                acc = acc + adjc[:, t * 8 + j:t * 8 + j + 1] * x3[j]
            out.append(acc)
        return out

    avenc = jnp.concatenate(adj_apply(av3), axis=0)
    enc_pre = _leaky(_dot(avenc, wfav_ref[...]) + _dot(ohbd, wfb2_ref[...])
                     + bias(BR_F, FEN))
    enc_all = jnp.concatenate(adj_apply(enc_pre.reshape(T, m, FEN)), axis=0)

    lat = _leaky(_dot(z_ref[...], latw_ref[...]) + bias(BR_LAT, 32))
    gi0c = _dot(lat, wih0l_ref[...]) + bias(BR_IH0, 96)
    gi0 = _dot(enc_all, wih0e_ref[...]).reshape(T, m, 96)

    whh0 = whh0_ref[...]
    whh1 = whh1_ref[...]
    wih1 = wih1_ref[...]
    bhh0 = bias(BR_HH0, 96)
    bhh1 = bias(BR_HH1, 96)
    bih1 = bias(BR_IH1, 96)

    def cell(h, git, whh, bhh):
        gh = _dot(h, whh) + bhh
        rz = jax.nn.sigmoid(git[:, :2 * H] + gh[:, :2 * H])
        r = rz[:, :H]
        zg = rz[:, H:]
        n = jnp.tanh(git[:, 2 * H:] + r * gh[:, 2 * H:])
        return (1.0 - zg) * n + zg * h

    h1 = jnp.zeros((m, H), jnp.float32)
    h2 = jnp.zeros((m, H), jnp.float32)
    dec = []
    for t in range(T):
        h1 = cell(h1, gi0[t] + gi0c, whh0, bhh0)
        h2 = cell(h2, _dot(h1, wih1) + bih1, whh1, bhh1)
        dec.append(h2)

    decall = jnp.concatenate(dec, axis=0)
    qkv = _dot(decall, wqkv_ref[...]) + bias(BR_QKV, 96)
    scale = 1.0 / float(DH) ** 0.5
    qall = qkv[:, 0:H] * scale
    qkv3 = qkv.reshape(T, m, 96)
    kcat = jnp.concatenate([qkv3[t][:, H:2 * H] for t in range(T)], axis=1)
    vcat = jnp.concatenate([qkv3[t][:, 2 * H:3 * H] for t in range(T)], axis=1)
    kcat_rep = jnp.concatenate([kcat] * T, axis=0)
    vcat_rep = jnp.concatenate([vcat] * T, axis=0)
    qrep = jnp.concatenate([qall] * T, axis=1)

    rr = jax.lax.broadcasted_iota(jnp.int32, (T * H, 128), 0)
    cc = jax.lax.broadcasted_iota(jnp.int32, (T * H, 128), 1)
    sel = jnp.where(cc == ((rr % H) // DH) * 8 + rr // H, 1.0, 0.0)
    s_all = _dot(qrep * kcat_rep, sel)

    ccol = jax.lax.broadcasted_iota(jnp.int32, (1, 128), 1)
    mask = jnp.concatenate(
        [jnp.broadcast_to(
            jnp.where((ccol < 16) & ((ccol % 8) <= i), 1.0, 0.0), (m, 128))
         for i in range(T)], axis=0)
    e_all = jnp.exp(s_all) * mask
    s0 = jnp.sum(e_all[:, 0:8], axis=1, keepdims=True)
    s1 = jnp.sum(e_all[:, 8:16], axis=1, keepdims=True)

    rr2 = jax.lax.broadcasted_iota(jnp.int32, (DH, 256), 0)
    cc2 = jax.lax.broadcasted_iota(jnp.int32, (DH, 256), 1)
    expand = jnp.where(rr2 == ((cc2 % H) // DH) * 8 + cc2 // H, 1.0, 0.0)
    pexp = _dot(e_all[:, 0:DH], expand)
    rr3 = jax.lax.broadcasted_iota(jnp.int32, (T * H, H), 0)
    cc3 = jax.lax.broadcasted_iota(jnp.int32, (T * H, H), 1)
    sumsel = jnp.where(rr3 % H == cc3, 1.0, 0.0)
    ctxw = _dot(pexp * vcat_rep, sumsel)
    rcat = jnp.concatenate([jnp.broadcast_to(1.0 / s0, (tm, DH)),
                            jnp.broadcast_to(1.0 / s1, (tm, DH))], axis=1)
    att = ctxw * rcat

    o = _dot(att, wo_ref[...]) + bias(BR_O, H)
    head = _dot(o, whead_ref[...]) + bias(BR_HEAD, 128)
    oav_ref[...] = head.reshape(T, m, 128)[:, :, 0:24]

    lrep = jnp.concatenate([head[:, 32:48]] * T, axis=1)
    h3 = head.reshape(T, m, 128)
    rcat2 = jnp.concatenate([h3[j][:, 48:64] for j in range(T)], axis=1)
    rrep = jnp.concatenate([rcat2] * T, axis=0)
    pairs = _leaky(lrep + rrep)
    obond_ref[...] = _dot(pairs, wbd_ref[...]).reshape(T, m, 40)


def kernel(atom_embed, atom_enc_w, atom_enc_b, bond_type_embed, valence_embed,
           valence_enc_w, valence_enc_b, final_enc_w, final_enc_b, lat_fc_w,
           lat_fc_scale, lat_fc_shift, gru0_wih, gru0_whh, gru0_bih, gru0_bhh,
           gru1_wih, gru1_whh, gru1_bih, gru1_bhh, attn0_wqkv, attn0_bqkv,
           attn0_wo, attn0_bo, bp_wl, bp_wr, bp_wo, val_out_w, val_out_b,
           atom_out_w, atom_out_b, z, atypes, bond_types, bonded_atoms,
           atom_valences, adj):
    f32 = jnp.float32
    B = z.shape[0]
    m = 512 if B % 512 == 0 else B

    at_i = atypes.astype(jnp.int32)
    bt_i = bond_types.reshape(B, 32).astype(jnp.int32)
    va_i = atom_valences.astype(jnp.int32)
    adj2 = adj.reshape(B, T * T)

    wav = (jnp.zeros((24, 96), f32)
           .at[:16, :AEN].set(atom_enc_w)
           .at[16:, AEN:AEN + VEN].set(valence_enc_w))
    wemb_av = (jnp.zeros((21, 24), f32)
               .at[0:16, 0:16].set(atom_embed)
               .at[16:21, 16:24].set(valence_embed))
    wav2 = wemb_av @ wav
    wfav = (jnp.zeros((96, FEN), f32)
            .at[:AEN].set(final_enc_w[:AEN])
            .at[AEN:AEN + VEN].set(final_enc_w[AEN + BEMB:]))
    wfb = final_enc_w[AEN:AEN + BEMB]
    wemb_b = jnp.zeros((20, BEMB), f32)
    for s in range(4):
        wemb_b = wemb_b.at[5 * s:5 * (s + 1), 8 * s:8 * (s + 1)].set(bond_type_embed)
    wfb2 = wemb_b @ wfb
    latw = lat_fc_w * lat_fc_scale[None, :]
    whead = (jnp.zeros((H, 128), f32)
             .at[:, 0:NA].set(atom_out_w)
             .at[:, NA:NA + NV].set(val_out_w)
             .at[:, 32:48].set(bp_wl)
             .at[:, 48:64].set(bp_wr))
    wbd = jnp.zeros((T * F, 40), f32)
    for j in range(T):
        wbd = wbd.at[j * F:(j + 1) * F, j * NB:(j + 1) * NB].set(bp_wo)

    bias = jnp.zeros((16, 128), f32)
    bias = bias.at[BR_AV, 0:AEN].set(atom_enc_b)
    bias = bias.at[BR_AV, AEN:AEN + VEN].set(valence_enc_b)
    bias = bias.at[BR_F, 0:FEN].set(final_enc_b)
    bias = bias.at[BR_LAT, 0:32].set(lat_fc_shift)
    bias = bias.at[BR_IH0, 0:96].set(gru0_bih)
    bias = bias.at[BR_HH0, 0:96].set(gru0_bhh)
    bias = bias.at[BR_IH1, 0:96].set(gru1_bih)
    bias = bias.at[BR_HH1, 0:96].set(gru1_bhh)
    bias = bias.at[BR_QKV, 0:96].set(attn0_bqkv)
    bias = bias.at[BR_O, 0:H].set(attn0_bo)
    bias = bias.at[BR_HEAD, 0:NA].set(atom_out_b)
    bias = bias.at[BR_HEAD, NA:NA + NV].set(val_out_b)

    body = functools.partial(_body, m=m)

    def wfull(shape):
        return pl.BlockSpec(shape, lambda i: (0, 0))

    oav, obond = pl.pallas_call(
        body,
        grid=(B // m,),
        in_specs=[
            pl.BlockSpec((m, T), lambda i: (i, 0)),
            pl.BlockSpec((m, 4 * T), lambda i: (i, 0)),
            pl.BlockSpec((m, T), lambda i: (i, 0)),
            pl.BlockSpec((m, LAT), lambda i: (i, 0)),
            pl.BlockSpec((m, T * T), lambda i: (i, 0)),
            wfull((21, 96)),
            wfull((96, FEN)),
            wfull((20, FEN)),
            wfull((LAT, 32)),
            wfull((32, 96)),
            wfull((32, 96)),
            wfull((32, 96)),
            wfull((32, 96)),
            wfull((32, 96)),
            wfull((32, 96)),
            wfull((32, 32)),
            wfull((H, 128)),
            wfull((T * F, 40)),
            wfull((16, 128)),
        ],
        out_specs=[pl.BlockSpec((T, m, 24), lambda i: (0, i, 0)),
                   pl.BlockSpec((T, m, 40), lambda i: (0, i, 0))],
        out_shape=(jax.ShapeDtypeStruct((T, B, 24), f32),
                   jax.ShapeDtypeStruct((T, B, 40), f32)),
        compiler_params=pltpu.CompilerParams(dimension_semantics=("parallel",)),
    )(at_i, bt_i, va_i, z, adj2, wav2, wfav, wfb2, latw,
      gru0_wih[:32], gru0_wih[32:], gru0_whh, gru1_wih, gru1_whh,
      attn0_wqkv, attn0_wo, whead, wbd, bias)

    av = oav.transpose(1, 0, 2)
    bonds = obond.transpose(1, 0, 2).reshape(B, T, T, NB)
    return {"atom_types": av[:, :, 0:NA],
            "bonds": bonds,
            "atom_valences": av[:, :, NA:NA + NV],
            "kps_1h": None}
